# baseline (speedup 1.0000x reference)
import functools

import jax
import jax.numpy as jnp
from jax import lax
from jax.experimental import pallas as pl
from jax.experimental.pallas import tpu as pltpu
from jax.experimental.pallas import tpu_sc as plsc

B = 1024
V = 100000
L = 50

NC = 2
NS = 16
NW = NC * NS
PER_W = B * L // NW
CHUNK = 128
LANES = 16

_CHUNKS = []
_off = 0
while _off < PER_W:
    _c = min(CHUNK, PER_W - _off)
    _CHUNKS.append((_off, _c))
    _off += _c

_PIECES = []
for _p in range(0, len(_CHUNKS), 4):
    grp = _CHUNKS[_p:_p + 4]
    _PIECES.append((grp[0][0], sum(c for _, c in grp), grp))

_mesh = plsc.VectorSubcoreMesh(core_axis_name="c", subcore_axis_name="s")


@functools.partial(
    pl.kernel,
    out_type=jax.ShapeDtypeStruct((B * L,), jnp.float32),
    mesh=_mesh,
    scratch_types=[
        pltpu.VMEM((PER_W,), jnp.int32),
        pltpu.VMEM((PER_W,), jnp.float32),
        pltpu.SemaphoreType.DMA,
        pltpu.SemaphoreType.DMA,
        pltpu.SemaphoreType.DMA,
    ],
)
def _pg_gather(idx_hbm, pred_hbm, out_hbm, idx_v, val_v, sem_in, sem_g, sem_o):
    wid = lax.axis_index("s") * NC + lax.axis_index("c")
    base = wid * PER_W
    in_cps = [
        pltpu.async_copy(idx_hbm.at[pl.ds(base + o, n)],
                         idx_v.at[pl.ds(o, n)], sem_in)
        for o, n, _ in _PIECES
    ]
    g_cps = []
    for (_, _, grp), icp in zip(_PIECES, in_cps):
        icp.wait()
        for o, c in grp:
            g_cps.append(
                pltpu.async_copy(pred_hbm.at[idx_v.at[pl.ds(o, c)]],
                                 val_v.at[pl.ds(o, c)], sem_g)
            )
    out_cps = []
    for (o, c), gcp in zip(_CHUNKS, g_cps):
        gcp.wait()
        out_cps.append(
            pltpu.async_copy(val_v.at[pl.ds(o, c)],
                             out_hbm.at[pl.ds(base + o, c)], sem_o)
        )
    for ocp in out_cps:
        ocp.wait()


def kernel(pred, target, reward):
    t = target.astype(jnp.int32)
    i = jnp.arange(B, dtype=jnp.int32)[:, None]
    n = ((t >> 3) << 13) + ((i >> 7) << 10) + ((t & 7) << 7) + (i & 127)
    pred_lin = pred.reshape(8, 128, V // 8, 8).transpose(2, 0, 3, 1).reshape(-1)
    val = _pg_gather(n.reshape(-1), pred_lin)
    return jnp.sum(val * reward.reshape(-1)) * jnp.float32(-1.0 / B)


# speedup vs baseline: 1.0015x; 1.0015x over previous
import functools

import jax
import jax.numpy as jnp
from jax import lax
from jax.experimental import pallas as pl
from jax.experimental.pallas import tpu as pltpu
from jax.experimental.pallas import tpu_sc as plsc

B = 1024
V = 100000
L = 50
LP = 64
TOT = B * LP

NC = 2
NS = 16
NW = NC * NS
MAIN_W = 49152 // NW
TAIL_W = 2048 // NW
PER_W = MAIN_W + TAIL_W
CHUNK = 128
LANES = 16

_mesh = plsc.VectorSubcoreMesh(core_axis_name="c", subcore_axis_name="s")


@functools.partial(
    pl.kernel,
    out_type=jax.ShapeDtypeStruct((TOT,), jnp.float32),
    mesh=_mesh,
    scratch_types=[
        pltpu.VMEM((PER_W,), jnp.int32),
        pltpu.VMEM((PER_W,), jnp.float32),
        pltpu.SemaphoreType.DMA,
        pltpu.SemaphoreType.DMA,
    ],
)
def _pg_gather(idx_hbm, pred_hbm, out_hbm, idx_v, val_v, sem_in, sem_g):
    wid = lax.axis_index("s") * NC + lax.axis_index("c")
    base = wid * MAIN_W
    tbase = 49152 + (wid >> 2) * 1024 + (wid & 3) * TAIL_W
    tcp = pltpu.async_copy(idx_hbm.at[pl.ds(tbase, TAIL_W)],
                           idx_v.at[pl.ds(MAIN_W, TAIL_W)], sem_in)
    pltpu.sync_copy(idx_hbm.at[pl.ds(base, MAIN_W)], idx_v.at[pl.ds(0, MAIN_W)])
    copies = [
        pltpu.async_copy(pred_hbm.at[idx_v.at[pl.ds(o, CHUNK)]],
                         val_v.at[pl.ds(o, CHUNK)], sem_g)
        for o in range(0, MAIN_W, CHUNK)
    ]
    tcp.wait()
    copies.append(
        pltpu.async_copy(pred_hbm.at[idx_v.at[pl.ds(MAIN_W, TAIL_W)]],
                         val_v.at[pl.ds(MAIN_W, TAIL_W)], sem_g)
    )
    for cp in copies:
        cp.wait()
    pltpu.sync_copy(val_v.at[pl.ds(0, MAIN_W)], out_hbm.at[pl.ds(base, MAIN_W)])
    pltpu.sync_copy(val_v.at[pl.ds(MAIN_W, TAIL_W)],
                    out_hbm.at[pl.ds(tbase, TAIL_W)])


def kernel(pred, target, reward):
    t = target.astype(jnp.int32)
    i = jnp.arange(B, dtype=jnp.int32)[:, None]
    n = ((t >> 3) << 13) + ((i >> 7) << 10) + ((t & 7) << 7) + (i & 127)
    n64 = jnp.pad(n, ((0, 0), (0, LP - L)))
    idx_lin = n64.reshape(8, 128, 8, 8).transpose(2, 0, 3, 1).reshape(-1)
    pred_lin = pred.reshape(8, 128, V // 8, 8).transpose(2, 0, 3, 1).reshape(-1)
    val = _pg_gather(idx_lin, pred_lin)
    val_view = val.reshape(8, 8, 8, 128).transpose(1, 3, 0, 2).reshape(B, LP)
    return jnp.sum(val_view[:, :L] * reward) * jnp.float32(-1.0 / B)


# speedup vs baseline: 1.0065x; 1.0050x over previous
import functools

import jax
import jax.numpy as jnp
from jax import lax
from jax.experimental import pallas as pl
from jax.experimental.pallas import tpu as pltpu
from jax.experimental.pallas import tpu_sc as plsc

B = 1024
V = 100000
L = 50

NC = 2
NS = 16
NW = NC * NS
PER_W = B * L // NW
CHUNK = 128
HALF = 800

_CHUNKS = []
_off = 0
while _off < PER_W:
    _c = min(CHUNK, PER_W - _off)
    _CHUNKS.append((_off, _c))
    _off += _c

_H1 = 768
_H2 = PER_W - _H1

_mesh = plsc.VectorSubcoreMesh(core_axis_name="c", subcore_axis_name="s")


@functools.partial(
    pl.kernel,
    out_type=jax.ShapeDtypeStruct((B * L,), jnp.float32),
    mesh=_mesh,
    scratch_types=[
        pltpu.VMEM((PER_W,), jnp.int32),
        pltpu.VMEM((PER_W,), jnp.float32),
        pltpu.SemaphoreType.DMA,
        pltpu.SemaphoreType.DMA,
        pltpu.SemaphoreType.DMA,
    ],
)
def _pg_gather(idx_hbm, pred_hbm, out_hbm, idx_v, val_v, sem_in, sem_g, sem_o):
    wid = lax.axis_index("s") * NC + lax.axis_index("c")
    base = wid * PER_W
    cp1 = pltpu.async_copy(idx_hbm.at[pl.ds(base, _H1)],
                           idx_v.at[pl.ds(0, _H1)], sem_in)
    cp2 = pltpu.async_copy(idx_hbm.at[pl.ds(base + _H1, _H2)],
                           idx_v.at[pl.ds(_H1, _H2)], sem_in)
    cp1.wait()
    g_cps = [
        pltpu.async_copy(pred_hbm.at[idx_v.at[pl.ds(o, c)]],
                         val_v.at[pl.ds(o, c)], sem_g)
        for o, c in _CHUNKS if o < _H1
    ]
    cp2.wait()
    g_cps += [
        pltpu.async_copy(pred_hbm.at[idx_v.at[pl.ds(o, c)]],
                         val_v.at[pl.ds(o, c)], sem_g)
        for o, c in _CHUNKS if o >= _H1
    ]
    for cp in g_cps[:6]:
        cp.wait()
    o1 = pltpu.async_copy(val_v.at[pl.ds(0, _H1)],
                          out_hbm.at[pl.ds(base, _H1)], sem_o)
    for cp in g_cps[6:]:
        cp.wait()
    o2 = pltpu.async_copy(val_v.at[pl.ds(_H1, _H2)],
                          out_hbm.at[pl.ds(base + _H1, _H2)], sem_o)
    o1.wait()
    o2.wait()


def kernel(pred, target, reward):
    t = target.astype(jnp.int32)
    i = jnp.arange(B, dtype=jnp.int32)[:, None]
    n = ((t >> 3) << 13) + ((i >> 7) << 10) + ((t & 7) << 7) + (i & 127)
    pred_lin = pred.reshape(8, 128, V // 8, 8).transpose(2, 0, 3, 1).reshape(-1)
    val = _pg_gather(n.reshape(-1), pred_lin)
    return jnp.sum(val * reward.reshape(-1)) * jnp.float32(-1.0 / B)
